# baseline (speedup 1.0000x reference)
import functools

import jax
import jax.numpy as jnp
from jax import lax
from jax.experimental import pallas as pl
from jax.experimental.pallas import tpu as pltpu
from jax.experimental.pallas import tpu_sc as plsc

BATCH = 16384
MF_DIM = 8
MLP_HALF = 32
NROWS = 1000000
K_BLK = 4
N_CMP = NROWS // K_BLK
BPREP = 1000
NUM_CORES = 2
NUM_SUBCORES = 16
NW = NUM_CORES * NUM_SUBCORES
BPW = BATCH // NW
CHUNK = 128
NCH = BPW // CHUNK

_f32 = jnp.float32


def _prep_body_mlp(a, b, c, d, out):
    out[:, 0:32] = a[...]
    out[:, 32:64] = b[...]
    out[:, 64:96] = c[...]
    out[:, 96:128] = d[...]


def _prep_body_gmf(a, b, c, d, out):
    pad = jnp.zeros((BPREP, MLP_HALF - MF_DIM), _f32)
    out[:, 0:32] = jnp.concatenate([a[...], pad], axis=1)
    out[:, 32:64] = jnp.concatenate([b[...], pad], axis=1)
    out[:, 64:96] = jnp.concatenate([c[...], pad], axis=1)
    out[:, 96:128] = jnp.concatenate([d[...], pad], axis=1)


def _compact(tbl, d, body):
    grid = N_CMP // BPREP
    specs = [
        pl.BlockSpec((BPREP, d), functools.partial(lambda i, kk: (i + grid * kk, 0), kk=k))
        for k in range(K_BLK)
    ]
    return pl.pallas_call(
        body,
        grid=(grid,),
        in_specs=specs,
        out_specs=pl.BlockSpec((BPREP, 128), lambda i: (i, 0)),
        out_shape=jax.ShapeDtypeStruct((N_CMP, 128), _f32),
    )(*([tbl] * K_BLK))


def _gather_table(tbl, hi, out, rb, sems, sem_w, base):
    def g_desc(c):
        slot = c % 2
        return pltpu.make_async_copy(
            tbl.at[hi.at[pl.ds(c * CHUNK, CHUNK)]],
            rb.at[pl.ds(slot * CHUNK, CHUNK)],
            sems[slot],
        )

    def w_desc(c):
        slot = c % 2
        return pltpu.make_async_copy(
            rb.at[pl.ds(slot * CHUNK, CHUNK)],
            out.at[pl.ds(base + c * CHUNK, CHUNK)],
            sem_w[slot],
        )

    g_desc(0).start()
    g_desc(1).start()
    for c in range(NCH):
        g_desc(c).wait()
        w_desc(c).start()
        if c + 2 < NCH:
            w_desc(c).wait()
            g_desc(c + 2).start()
    for c in range(NCH - 2, NCH):
        w_desc(c).wait()


def _sc_body(uidx_hbm, iidx_hbm, cmu, cmi, cgu, cgi,
             out_mu, out_mi, out_gu, out_gi,
             hi_u, hi_i, rb,
             s0, s1, w0, w1):
    wid = lax.axis_index("s") * NUM_CORES + lax.axis_index("c")
    base = wid * BPW
    pltpu.sync_copy(uidx_hbm.at[pl.ds(base, BPW)], hi_u)
    pltpu.sync_copy(iidx_hbm.at[pl.ds(base, BPW)], hi_i)
    for s in range(BPW // 16):
        sl = pl.ds(s * 16, 16)
        hi_u[sl] = lax.shift_right_logical(hi_u[sl], 2)
        hi_i[sl] = lax.shift_right_logical(hi_i[sl], 2)
    sems = (s0, s1)
    sem_w = (w0, w1)
    _gather_table(cmu, hi_u, out_mu, rb, sems, sem_w, base)
    _gather_table(cmi, hi_i, out_mi, rb, sems, sem_w, base)
    for s in range(BPW // 16):
        sl = pl.ds(s * 16, 16)
        hi_u[sl] = lax.shift_right_logical(hi_u[sl], 2)
        hi_i[sl] = lax.shift_right_logical(hi_i[sl], 2)
    _gather_table(cgu, hi_u, out_gu, rb, sems, sem_w, base)
    _gather_table(cgi, hi_i, out_gi, rb, sems, sem_w, base)


@functools.cache
def _sc_gather():
    return functools.partial(
        pl.kernel,
        out_type=tuple(
            jax.ShapeDtypeStruct((BATCH, 128), _f32) for _ in range(4)),
        mesh=plsc.VectorSubcoreMesh(core_axis_name="c", subcore_axis_name="s"),
        scratch_types=[
            pltpu.VMEM((BPW,), jnp.int32),
            pltpu.VMEM((BPW,), jnp.int32),
            pltpu.VMEM((2 * CHUNK, 128), _f32),
        ] + [pltpu.SemaphoreType.DMA] * 4,
        compiler_params=pltpu.CompilerParams(use_tc_tiling_on_sc=True),
    )(_sc_body)


BM = 2048


def _mlp_body(uidx, iidx, rmu, rmi, rgu, rgi,
              w1a, w1b, b1, w2, b2, w3, b3, wpg, wph, bp, out):
    u = uidx[...]
    v = iidx[...]

    def pick(raw, key, d):
        nblk = 128 // d
        k_id = jnp.bitwise_and(key, nblk - 1)
        acc = jnp.where(k_id == 0, raw[:, 0:d], 0.0)
        for k in range(1, nblk):
            acc = acc + jnp.where(k_id == k, raw[:, d * k:d * (k + 1)], 0.0)
        return acc

    mu = pick(rmu[...], u, MLP_HALF)
    mi = pick(rmi[...], v, MLP_HALF)
    gu = pick(rgu[...], u, MF_DIM)
    gi = pick(rgi[...], v, MF_DIM)
    h = jnp.maximum(mu @ w1a[...] + mi @ w1b[...] + b1[...], 0.0)
    h = jnp.maximum(h @ w2[...] + b2[...], 0.0)
    h = jnp.maximum(h @ w3[...] + b3[...], 0.0)
    g = gu * gi
    logit = g @ wpg[...] + h @ wph[...] + bp[...]
    out[...] = jax.nn.sigmoid(logit[:, 0])


def kernel(user_indices, item_indices, ue_gmf, ie_gmf, ue_mlp, ie_mlp,
           W1, b1, W2, b2, W3, b3, Wp, bp):
    uidx = user_indices.astype(jnp.int32)
    iidx = item_indices.astype(jnp.int32)
    zero = lax.optimization_barrier(jnp.float32(0.0))
    cmu = ue_mlp.reshape(NROWS // 4, 128) + zero
    cmi = ie_mlp.reshape(NROWS // 4, 128) + zero
    cgu = ue_gmf.reshape(NROWS // 16, 128) + zero
    cgi = ie_gmf.reshape(NROWS // 16, 128) + zero
    rmu, rmi, rgu, rgi = _sc_gather()(uidx, iidx, cmu, cmi, cgu, cgi)

    grid = BATCH // BM
    raw_spec = pl.BlockSpec((BM, 128), lambda i: (i, 0))
    vec = pl.BlockSpec((BM, 1), lambda i: (i, 0))
    uidx2 = uidx.reshape(BATCH, 1)
    iidx2 = iidx.reshape(BATCH, 1)
    full = lambda a: pl.BlockSpec(a.shape, lambda i: (0,) * a.ndim)
    w1a, w1b = W1[:MLP_HALF], W1[MLP_HALF:]
    wpg, wph = Wp[:MF_DIM], Wp[MF_DIM:]
    b1r, b2r, b3r, bpr = (b.reshape(1, -1) for b in (b1, b2, b3, bp))
    out = pl.pallas_call(
        _mlp_body,
        grid=(grid,),
        in_specs=[vec, vec, raw_spec, raw_spec, raw_spec, raw_spec,
                  full(w1a), full(w1b), full(b1r), full(W2), full(b2r),
                  full(W3), full(b3r), full(wpg), full(wph), full(bpr)],
        out_specs=pl.BlockSpec((BM,), lambda i: (i,)),
        out_shape=jax.ShapeDtypeStruct((BATCH,), _f32),
    )(uidx2, iidx2, rmu, rmi, rgu, rgi,
      w1a, w1b, b1r, W2, b2r, W3, b3r, wpg, wph, bpr)
    return out


# speedup vs baseline: 2.4962x; 2.4962x over previous
import functools

import jax
import jax.numpy as jnp
from jax import lax
from jax.experimental import pallas as pl
from jax.experimental.pallas import tpu as pltpu
from jax.experimental.pallas import tpu_sc as plsc

BATCH = 16384
MF_DIM = 8
MLP_HALF = 32
NUM_CORES = 2
NUM_SUBCORES = 16
NW = NUM_CORES * NUM_SUBCORES
BPW = BATCH // NW
CHUNK = 128
NCH = BPW // CHUNK

_f32 = jnp.float32


def _gather_table(tbl, idx_v, out, rb, sems, sem_w, base):
    def issue(c):
        slot = c % 2

        def body(j, _):
            u = idx_v[pl.ds(c * CHUNK + j, 16)][0]
            pltpu.make_async_copy(
                tbl.at[pl.ds(u, 1)],
                rb.at[pl.ds(slot * CHUNK + j, 1)],
                sems[slot],
            ).start()
            return _
        lax.fori_loop(0, CHUNK, body, None)

    def drain(c):
        def body(j, _):
            pltpu.make_async_copy(
                tbl.at[pl.ds(0, 1)], rb.at[pl.ds(0, 1)], sems[c % 2]
            ).wait()
            return _
        lax.fori_loop(0, CHUNK, body, None)

    def w_desc(c):
        slot = c % 2
        return pltpu.make_async_copy(
            rb.at[pl.ds(slot * CHUNK, CHUNK)],
            out.at[pl.ds(base + c * CHUNK, CHUNK)],
            sem_w[slot],
        )

    issue(0)
    issue(1)
    for c in range(NCH):
        drain(c)
        w_desc(c).start()
        if c + 2 < NCH:
            w_desc(c).wait()
            issue(c + 2)
    for c in range(NCH - 2, NCH):
        w_desc(c).wait()


def _sc_body(uidx_hbm, iidx_hbm, mu2, mi2, gu2, gi2,
             out_mu, out_mi, out_gu, out_gi,
             uidx_v, iidx_v, rb, rbg,
             s0, s1, w0, w1):
    wid = lax.axis_index("s") * NUM_CORES + lax.axis_index("c")
    base = wid * BPW
    pltpu.sync_copy(uidx_hbm.at[pl.ds(base, BPW)], uidx_v.at[pl.ds(0, BPW)])
    pltpu.sync_copy(iidx_hbm.at[pl.ds(base, BPW)], iidx_v.at[pl.ds(0, BPW)])
    sems = (s0, s1)
    sem_w = (w0, w1)
    _gather_table(mu2, uidx_v, out_mu, rb, sems, sem_w, base)
    _gather_table(mi2, iidx_v, out_mi, rb, sems, sem_w, base)
    _gather_table(gu2, uidx_v, out_gu, rbg, sems, sem_w, base)
    _gather_table(gi2, iidx_v, out_gi, rbg, sems, sem_w, base)


@functools.cache
def _sc_gather():
    return functools.partial(
        pl.kernel,
        out_type=(
            jax.ShapeDtypeStruct((BATCH, MLP_HALF), _f32),
            jax.ShapeDtypeStruct((BATCH, MLP_HALF), _f32),
            jax.ShapeDtypeStruct((BATCH, MF_DIM), _f32),
            jax.ShapeDtypeStruct((BATCH, MF_DIM), _f32),
        ),
        mesh=plsc.VectorSubcoreMesh(core_axis_name="c", subcore_axis_name="s"),
        scratch_types=[
            pltpu.VMEM((BPW + 16,), jnp.int32),
            pltpu.VMEM((BPW + 16,), jnp.int32),
            pltpu.VMEM((2 * CHUNK, MLP_HALF), _f32),
            pltpu.VMEM((2 * CHUNK, MF_DIM), _f32),
        ] + [pltpu.SemaphoreType.DMA] * 4,
        compiler_params=pltpu.CompilerParams(use_tc_tiling_on_sc=True),
    )(_sc_body)


BM = 2048


def _mlp_body(mu, mi, gu, gi, w1a, w1b, b1, w2, b2, w3, b3, wpg, wph, bp, out):
    h = jnp.maximum(mu[...] @ w1a[...] + mi[...] @ w1b[...] + b1[...], 0.0)
    h = jnp.maximum(h @ w2[...] + b2[...], 0.0)
    h = jnp.maximum(h @ w3[...] + b3[...], 0.0)
    g = gu[...] * gi[...]
    logit = g @ wpg[...] + h @ wph[...] + bp[...]
    out[...] = jax.nn.sigmoid(logit[:, 0])


def kernel(user_indices, item_indices, ue_gmf, ie_gmf, ue_mlp, ie_mlp,
           W1, b1, W2, b2, W3, b3, Wp, bp):
    uidx = user_indices.astype(jnp.int32)
    iidx = item_indices.astype(jnp.int32)
    mu, mi, gu, gi = _sc_gather()(uidx, iidx, ue_mlp, ie_mlp, ue_gmf, ie_gmf)

    grid = BATCH // BM
    row_spec = lambda d: pl.BlockSpec((BM, d), lambda i: (i, 0))
    full = lambda a: pl.BlockSpec(a.shape, lambda i: (0,) * a.ndim)
    w1a, w1b = W1[:MLP_HALF], W1[MLP_HALF:]
    wpg, wph = Wp[:MF_DIM], Wp[MF_DIM:]
    b1r, b2r, b3r, bpr = (b.reshape(1, -1) for b in (b1, b2, b3, bp))
    out = pl.pallas_call(
        _mlp_body,
        grid=(grid,),
        in_specs=[row_spec(MLP_HALF), row_spec(MLP_HALF),
                  row_spec(MF_DIM), row_spec(MF_DIM),
                  full(w1a), full(w1b), full(b1r), full(W2), full(b2r),
                  full(W3), full(b3r), full(wpg), full(wph), full(bpr)],
        out_specs=pl.BlockSpec((BM,), lambda i: (i,)),
        out_shape=jax.ShapeDtypeStruct((BATCH,), _f32),
    )(mu, mi, gu, gi, w1a, w1b, b1r, W2, b2r, W3, b3r, wpg, wph, bpr)
    return out


# speedup vs baseline: 2.5349x; 1.0155x over previous
import functools

import jax
import jax.numpy as jnp
from jax import lax
from jax.experimental import pallas as pl
from jax.experimental.pallas import tpu as pltpu
from jax.experimental.pallas import tpu_sc as plsc

BATCH = 16384
MF_DIM = 8
MLP_HALF = 32
NUM_CORES = 2
NUM_SUBCORES = 16
NW = NUM_CORES * NUM_SUBCORES
BPW = BATCH // NW
CHUNK = 128
NCH = BPW // CHUNK

_f32 = jnp.float32


def _gather_table(tbl, idx_v, out, rb, sems, sem_w, base):
    def issue(c):
        slot = c % 2

        def body(j, _):
            u = idx_v[pl.ds(c * CHUNK + j, 16)][0]
            pltpu.make_async_copy(
                tbl.at[pl.ds(u, 1)],
                rb.at[pl.ds(slot * CHUNK + j, 1)],
                sems[slot],
            ).start()
            return _
        lax.fori_loop(0, CHUNK, body, None)

    def drain(c):
        def body(j, _):
            pltpu.make_async_copy(
                tbl.at[pl.ds(0, 1)], rb.at[pl.ds(0, 1)], sems[c % 2]
            ).wait()
            return _
        lax.fori_loop(0, CHUNK, body, None)

    def w_desc(c):
        slot = c % 2
        return pltpu.make_async_copy(
            rb.at[pl.ds(slot * CHUNK, CHUNK)],
            out.at[pl.ds(base + c * CHUNK, CHUNK)],
            sem_w[slot],
        )

    issue(0)
    issue(1)
    for c in range(NCH):
        drain(c)
        w_desc(c).start()
        if c + 2 < NCH:
            w_desc(c).wait()
            issue(c + 2)
    for c in range(NCH - 2, NCH):
        w_desc(c).wait()


def _sc_body(uidx_hbm, iidx_hbm, tu2, ti2,
             out_u, out_i,
             uidx_v, iidx_v, rb,
             s0, s1, w0, w1):
    wid = lax.axis_index("s") * NUM_CORES + lax.axis_index("c")
    base = wid * BPW
    pltpu.sync_copy(uidx_hbm.at[pl.ds(base, BPW)], uidx_v.at[pl.ds(0, BPW)])
    pltpu.sync_copy(iidx_hbm.at[pl.ds(base, BPW)], iidx_v.at[pl.ds(0, BPW)])
    sems = (s0, s1)
    sem_w = (w0, w1)
    _gather_table(tu2, uidx_v, out_u, rb, sems, sem_w, base)
    _gather_table(ti2, iidx_v, out_i, rb, sems, sem_w, base)


@functools.cache
def _sc_gather(d):
    return functools.partial(
        pl.kernel,
        out_type=(
            jax.ShapeDtypeStruct((BATCH, d), _f32),
            jax.ShapeDtypeStruct((BATCH, d), _f32),
        ),
        mesh=plsc.VectorSubcoreMesh(core_axis_name="c", subcore_axis_name="s"),
        scratch_types=[
            pltpu.VMEM((BPW + 16,), jnp.int32),
            pltpu.VMEM((BPW + 16,), jnp.int32),
            pltpu.VMEM((2 * CHUNK, d), _f32),
        ] + [pltpu.SemaphoreType.DMA] * 4,
        compiler_params=pltpu.CompilerParams(use_tc_tiling_on_sc=True),
    )(_sc_body)


BM = 2048


def _mlp_body(mu, mi, gu, gi, w1a, w1b, b1, w2, b2, w3, b3, wpg, wph, bp, out):
    h = jnp.maximum(mu[...] @ w1a[...] + mi[...] @ w1b[...] + b1[...], 0.0)
    h = jnp.maximum(h @ w2[...] + b2[...], 0.0)
    h = jnp.maximum(h @ w3[...] + b3[...], 0.0)
    g = gu[...] * gi[...]
    logit = g @ wpg[...] + h @ wph[...] + bp[...]
    out[...] = jax.nn.sigmoid(logit[:, 0])


def kernel(user_indices, item_indices, ue_gmf, ie_gmf, ue_mlp, ie_mlp,
           W1, b1, W2, b2, W3, b3, Wp, bp):
    uidx = user_indices.astype(jnp.int32)
    iidx = item_indices.astype(jnp.int32)
    mu, mi = _sc_gather(MLP_HALF)(uidx, iidx, ue_mlp, ie_mlp)
    gu, gi = _sc_gather(MF_DIM)(uidx, iidx, ue_gmf, ie_gmf)

    grid = BATCH // BM
    row_spec = lambda d: pl.BlockSpec((BM, d), lambda i: (i, 0))
    full = lambda a: pl.BlockSpec(a.shape, lambda i: (0,) * a.ndim)
    w1a, w1b = W1[:MLP_HALF], W1[MLP_HALF:]
    wpg, wph = Wp[:MF_DIM], Wp[MF_DIM:]
    b1r, b2r, b3r, bpr = (b.reshape(1, -1) for b in (b1, b2, b3, bp))
    out = pl.pallas_call(
        _mlp_body,
        grid=(grid,),
        in_specs=[row_spec(MLP_HALF), row_spec(MLP_HALF),
                  row_spec(MF_DIM), row_spec(MF_DIM),
                  full(w1a), full(w1b), full(b1r), full(W2), full(b2r),
                  full(W3), full(b3r), full(wpg), full(wph), full(bpr)],
        out_specs=pl.BlockSpec((BM,), lambda i: (i,)),
        out_shape=jax.ShapeDtypeStruct((BATCH,), _f32),
    )(mu, mi, gu, gi, w1a, w1b, b1r, W2, b2r, W3, b3r, wpg, wph, bpr)
    return out
